# baseline (speedup 1.0000x reference)
import functools

import jax
import jax.numpy as jnp
from jax import lax
from jax.experimental import pallas as pl
from jax.experimental.pallas import tpu as pltpu
from jax.experimental.pallas import tpu_sc as plsc

N_TOKENS = 32768
D_EMBED = 768
N_EXPERTS = 64
BLOCK = 4096

NW = 32
TOK_PER_W = N_TOKENS // NW
GROUPS = TOK_PER_W // 16


def _logits_block(x_ref, wt_ref, b_ref, lt_ref):
    logits = jnp.dot(x_ref[...], wt_ref[...], preferred_element_type=jnp.float32)
    lt_ref[...] = (logits + b_ref[...]).T


_sc_mesh = plsc.VectorSubcoreMesh(core_axis_name="c", subcore_axis_name="s")


@functools.partial(
    pl.kernel,
    out_type=[
        jax.ShapeDtypeStruct((2, N_TOKENS), jnp.float32),
        jax.ShapeDtypeStruct((2, N_TOKENS), jnp.int32),
    ],
    mesh=_sc_mesh,
    scratch_types=[
        pltpu.VMEM((N_EXPERTS, TOK_PER_W), jnp.float32),
        pltpu.VMEM((2, TOK_PER_W), jnp.float32),
        pltpu.VMEM((2, TOK_PER_W), jnp.int32),
    ],
)
def _sc_top2(lt_hbm, s_hbm, i_hbm, buf, sbuf, ibuf):
    wid = lax.axis_index("s") * 2 + lax.axis_index("c")
    base = wid * TOK_PER_W
    pltpu.sync_copy(lt_hbm.at[:, pl.ds(base, TOK_PER_W)], buf)

    big = jnp.full((16,), 64.0, jnp.float32)
    neg = jnp.full((16,), -jnp.inf, jnp.float32)

    @pl.loop(0, GROUPS)
    def _grp(g):
        t = g * 16

        def le(e):
            return buf[e, pl.ds(t, 16)]

        m1 = le(0)
        for e in range(1, N_EXPERTS):
            m1 = jnp.maximum(m1, le(e))
        i1 = big
        for e in range(N_EXPERTS):
            ef = jnp.full((16,), float(e), jnp.float32)
            i1 = jnp.minimum(i1, jnp.where(le(e) == m1, ef, big))
        m2 = neg
        for e in range(N_EXPERTS):
            ef = jnp.full((16,), float(e), jnp.float32)
            m2 = jnp.maximum(m2, jnp.where(ef == i1, neg, le(e)))
        i2 = big
        for e in range(N_EXPERTS):
            ef = jnp.full((16,), float(e), jnp.float32)
            i2 = jnp.minimum(i2, jnp.where((le(e) == m2) & (ef != i1), ef, big))
        d = jnp.zeros((16,), jnp.float32)
        for e in range(N_EXPERTS):
            d = d + jnp.exp(le(e) - m1)
        s1 = 1.0 / d
        s2 = jnp.exp(m2 - m1) / d
        sbuf[0, pl.ds(t, 16)] = s1
        sbuf[1, pl.ds(t, 16)] = s2
        ibuf[0, pl.ds(t, 16)] = i1.astype(jnp.int32)
        ibuf[1, pl.ds(t, 16)] = i2.astype(jnp.int32)

    pltpu.sync_copy(sbuf, s_hbm.at[:, pl.ds(base, TOK_PER_W)])
    pltpu.sync_copy(ibuf, i_hbm.at[:, pl.ds(base, TOK_PER_W)])


@jax.jit
def kernel(x, W, b):
    wt = W.T
    b2 = b.reshape(1, N_EXPERTS)
    grid = (N_TOKENS // BLOCK,)
    lt = pl.pallas_call(
        _logits_block,
        grid=grid,
        in_specs=[
            pl.BlockSpec((BLOCK, D_EMBED), lambda i: (i, 0)),
            pl.BlockSpec((D_EMBED, N_EXPERTS), lambda i: (0, 0)),
            pl.BlockSpec((1, N_EXPERTS), lambda i: (0, 0)),
        ],
        out_specs=pl.BlockSpec((N_EXPERTS, BLOCK), lambda i: (0, i)),
        out_shape=jax.ShapeDtypeStruct((N_EXPERTS, N_TOKENS), jnp.float32),
    )(x, wt, b2)
    scores_t, idx_t = _sc_top2(lt)
    return scores_t.T, idx_t.T


# speedup vs baseline: 2.1365x; 2.1365x over previous
import jax
import jax.numpy as jnp
from jax.experimental import pallas as pl

N_TOKENS = 32768
D_EMBED = 768
N_EXPERTS = 64
BLOCK = 4096


def _router_block(x_ref, wt_ref, b_ref, scores_ref, idx_ref):
    x_blk = x_ref[...]
    logits = jnp.dot(x_blk, wt_ref[...], preferred_element_type=jnp.float32)
    logits = logits + b_ref[...]
    lt = logits.T

    eid = jax.lax.broadcasted_iota(jnp.int32, lt.shape, 0).astype(jnp.float32)
    m1 = jnp.max(lt, axis=0, keepdims=True)
    i1f = jnp.min(jnp.where(lt == m1, eid, 64.0), axis=0, keepdims=True)
    lt2 = jnp.where(eid == i1f, -jnp.inf, lt)
    m2 = jnp.max(lt2, axis=0, keepdims=True)
    i2f = jnp.min(jnp.where(lt2 == m2, eid, 64.0), axis=0, keepdims=True)

    denom = jnp.sum(jnp.exp(lt - m1), axis=0, keepdims=True)
    s1 = 1.0 / denom
    s2 = jnp.exp(m2 - m1) / denom

    scores_ref[...] = jnp.concatenate([s1, s2], axis=0)
    idx_ref[...] = jnp.concatenate([i1f, i2f], axis=0).astype(jnp.int32)


@jax.jit
def kernel(x, W, b):
    wt = W.T
    b2 = b.reshape(1, N_EXPERTS)
    grid = (N_TOKENS // BLOCK,)
    scores_t, idx_t = pl.pallas_call(
        _router_block,
        grid=grid,
        in_specs=[
            pl.BlockSpec((BLOCK, D_EMBED), lambda i: (i, 0)),
            pl.BlockSpec((D_EMBED, N_EXPERTS), lambda i: (0, 0)),
            pl.BlockSpec((1, N_EXPERTS), lambda i: (0, 0)),
        ],
        out_specs=[
            pl.BlockSpec((2, BLOCK), lambda i: (0, i)),
            pl.BlockSpec((2, BLOCK), lambda i: (0, i)),
        ],
        out_shape=[
            jax.ShapeDtypeStruct((2, N_TOKENS), jnp.float32),
            jax.ShapeDtypeStruct((2, N_TOKENS), jnp.int32),
        ],
    )(x, wt, b2)
    return scores_t.T, idx_t.T
